# baseline (speedup 1.0000x reference)
import functools

import jax
import jax.numpy as jnp
from jax import lax
from jax.experimental import pallas as pl
from jax.experimental.pallas import tpu as pltpu
from jax.experimental.pallas import tpu_sc as plsc

D = 128
TEMP_INV = 10.0
EPS = 1e-8
NC = 2
NS = 16
NW = NC * NS
L = 16
CHUNK = 128


def _fast_rsqrt(x):
    i = lax.bitcast_convert_type(x, jnp.int32)
    i = jnp.int32(0x5F3759DF) - lax.shift_right_arithmetic(i, 1)
    y = lax.bitcast_convert_type(i, jnp.float32)
    for _ in range(3):
        y = y * (1.5 - 0.5 * x * y * y)
    return y


def _make_sc_kernel(n_pairs, n_pos):
    assert n_pairs % (NW * CHUNK) == 0
    ppt = n_pairs // NW
    nch = ppt // CHUNK
    assert nch % 2 == 0
    mesh = plsc.VectorSubcoreMesh(core_axis_name="c", subcore_axis_name="s")

    @functools.partial(
        pl.kernel,
        mesh=mesh,
        compiler_params=pltpu.CompilerParams(
            needs_layout_passes=False, use_tc_tiling_on_sc=False),
        out_type=[
            jax.ShapeDtypeStruct((NW, L), jnp.float32),
            jax.ShapeDtypeStruct((NW, L), jnp.float32),
        ],
        scratch_types=[
            pltpu.VMEM((ppt,), jnp.int32),
            pltpu.VMEM((ppt,), jnp.int32),
            pltpu.VMEM((CHUNK, D), jnp.float32),
            pltpu.VMEM((CHUNK, D), jnp.float32),
            pltpu.VMEM((CHUNK, D), jnp.float32),
            pltpu.VMEM((CHUNK, D), jnp.float32),
            pltpu.VMEM((L, 17), jnp.float32),
            pltpu.VMEM((L, 17), jnp.float32),
            pltpu.VMEM((L, 17), jnp.float32),
            pltpu.VMEM((L,), jnp.float32),
            pltpu.VMEM((L,), jnp.float32),
            pltpu.SemaphoreType.DMA,
            pltpu.SemaphoreType.DMA,
            pltpu.SemaphoreType.DMA,
            pltpu.SemaphoreType.DMA,
        ],
    )
    def sc_kernel(z_hbm, ii_hbm, jj_hbm, oexp_hbm, opos_hbm,
                  ii_v, jj_v, ri0, ri1, rj0, rj1,
                  mab, maa, mbb, accexp, accpos, si0, si1, sj0, sj1):
        wid = lax.axis_index("s") * NC + lax.axis_index("c")
        base = wid * ppt
        pltpu.sync_copy(ii_hbm.at[pl.ds(base, ppt)], ii_v)
        pltpu.sync_copy(jj_hbm.at[pl.ds(base, ppt)], jj_v)
        accexp[...] = jnp.zeros((L,), jnp.float32)
        accpos[...] = jnp.zeros((L,), jnp.float32)
        lane = lax.broadcasted_iota(jnp.int32, (L,), 0)
        col15 = jnp.full((L,), L - 1, jnp.int32)
        bufs = ((ri0, rj0, si0, sj0), (ri1, rj1, si1, sj1))

        def issue(ch, b):
            ri, rj, si, sj = bufs[b]
            off = ch * CHUNK
            pltpu.async_copy(z_hbm.at[ii_v.at[pl.ds(off, CHUNK)]], ri, si)
            pltpu.async_copy(z_hbm.at[jj_v.at[pl.ds(off, CHUNK)]], rj, sj)

        def wait(b):
            ri, rj, si, sj = bufs[b]
            pltpu.make_async_copy(z_hbm.at[ii_v.at[pl.ds(0, CHUNK)]],
                                  ri, si).wait()
            pltpu.make_async_copy(z_hbm.at[jj_v.at[pl.ds(0, CHUNK)]],
                                  rj, sj).wait()

        def compute(ch, b):
            ri, rj, _, _ = bufs[b]
            off = ch * CHUNK

            def group_body(g, _):
                for k in range(L):
                    p = g * L + k
                    ab0 = jnp.zeros((L,), jnp.float32)
                    aa0 = jnp.zeros((L,), jnp.float32)
                    bb0 = jnp.zeros((L,), jnp.float32)
                    ab1 = jnp.zeros((L,), jnp.float32)
                    aa1 = jnp.zeros((L,), jnp.float32)
                    bb1 = jnp.zeros((L,), jnp.float32)
                    for s in range(D // (2 * L)):
                        av0 = ri[p, pl.ds(2 * s * L, L)]
                        bv0 = rj[p, pl.ds(2 * s * L, L)]
                        av1 = ri[p, pl.ds((2 * s + 1) * L, L)]
                        bv1 = rj[p, pl.ds((2 * s + 1) * L, L)]
                        ab0 = ab0 + av0 * bv0
                        aa0 = aa0 + av0 * av0
                        bb0 = bb0 + bv0 * bv0
                        ab1 = ab1 + av1 * bv1
                        aa1 = aa1 + av1 * av1
                        bb1 = bb1 + bv1 * bv1
                    mab[k, pl.ds(0, L)] = jnp.cumsum(ab0 + ab1)
                    maa[k, pl.ds(0, L)] = jnp.cumsum(aa0 + aa1)
                    mbb[k, pl.ds(0, L)] = jnp.cumsum(bb0 + bb1)
                ab_vec = plsc.load_gather(mab, [lane, col15])
                aa_vec = plsc.load_gather(maa, [lane, col15])
                bb_vec = plsc.load_gather(mbb, [lane, col15])
                inv = (_fast_rsqrt(jnp.maximum(aa_vec, EPS * EPS)) *
                       _fast_rsqrt(jnp.maximum(bb_vec, EPS * EPS)))
                logit = ab_vec * inv * TEMP_INV
                accexp[...] += jnp.exp(logit - 10.0)
                gidx = base + off + g * L + lane
                accpos[...] += jnp.where(gidx < n_pos, logit, 0.0)
                return 0

            lax.fori_loop(0, CHUNK // L, group_body, 0)

        issue(0, 0)

        def outer_body(kk, _):
            for b in (0, 1):
                ch = kk * 2 + b

                @pl.when(ch + 1 < nch)
                def _():
                    issue(ch + 1, 1 - b)

                wait(b)
                compute(ch, b)
            return 0

        lax.fori_loop(0, nch // 2, outer_body, 0)
        pltpu.sync_copy(accexp, oexp_hbm.at[wid])
        pltpu.sync_copy(accpos, opos_hbm.at[wid])

    return sc_kernel


def kernel(z_all, pos_pairs, neg_pairs):
    n_pos = pos_pairs.shape[0]
    pairs = jnp.concatenate([pos_pairs, neg_pairs], axis=0)
    ii = pairs[:, 0]
    jj = pairs[:, 1]
    sc = _make_sc_kernel(pairs.shape[0], n_pos)
    part_exp, part_pos = sc(z_all, ii, jj)
    lse = 10.0 + jnp.log(jnp.sum(part_exp))
    return lse - jnp.sum(part_pos) / n_pos


# speedup vs baseline: 1.6179x; 1.6179x over previous
import functools

import jax
import jax.numpy as jnp
from jax import lax
from jax.experimental import pallas as pl
from jax.experimental.pallas import tpu as pltpu
from jax.experimental.pallas import tpu_sc as plsc

D = 128
TEMP_INV = 10.0
EPS = 1e-8
NC = 2
NS = 16
NW = NC * NS
L = 16
CHUNK = 128


def _fast_rsqrt(x):
    i = lax.bitcast_convert_type(x, jnp.int32)
    i = jnp.int32(0x5F3759DF) - lax.shift_right_arithmetic(i, 1)
    y = lax.bitcast_convert_type(i, jnp.float32)
    for _ in range(3):
        y = y * (1.5 - 0.5 * x * y * y)
    return y


def _make_sc_kernel(n_pairs, n_pos):
    assert n_pairs % (NW * CHUNK) == 0
    ppt = n_pairs // NW
    nch = ppt // CHUNK
    assert nch % 2 == 0
    mesh = plsc.VectorSubcoreMesh(core_axis_name="c", subcore_axis_name="s")

    @functools.partial(
        pl.kernel,
        mesh=mesh,
        compiler_params=pltpu.CompilerParams(
            needs_layout_passes=False, use_tc_tiling_on_sc=False),
        out_type=[
            jax.ShapeDtypeStruct((NW, L), jnp.float32),
            jax.ShapeDtypeStruct((NW, L), jnp.float32),
        ],
        scratch_types=[
            pltpu.VMEM((ppt,), jnp.int32),
            pltpu.VMEM((ppt,), jnp.int32),
            pltpu.VMEM((CHUNK, D), jnp.float32),
            pltpu.VMEM((CHUNK, D), jnp.float32),
            pltpu.VMEM((CHUNK, D), jnp.float32),
            pltpu.VMEM((CHUNK, D), jnp.float32),
            pltpu.VMEM((L,), jnp.float32),
            pltpu.VMEM((L,), jnp.float32),
            pltpu.SemaphoreType.DMA,
            pltpu.SemaphoreType.DMA,
            pltpu.SemaphoreType.DMA,
            pltpu.SemaphoreType.DMA,
        ],
    )
    def sc_kernel(z_hbm, ii_hbm, jj_hbm, oexp_hbm, opos_hbm,
                  ii_v, jj_v, ri0, ri1, rj0, rj1,
                  accexp, accpos, si0, si1, sj0, sj1):
        wid = lax.axis_index("s") * NC + lax.axis_index("c")
        base = wid * ppt
        pltpu.sync_copy(ii_hbm.at[pl.ds(base, ppt)], ii_v)
        pltpu.sync_copy(jj_hbm.at[pl.ds(base, ppt)], jj_v)
        accexp[...] = jnp.zeros((L,), jnp.float32)
        accpos[...] = jnp.zeros((L,), jnp.float32)
        lane = lax.broadcasted_iota(jnp.int32, (L,), 0)
        bufs = ((ri0, rj0, si0, sj0), (ri1, rj1, si1, sj1))

        def issue(ch, b):
            ri, rj, si, sj = bufs[b]
            off = ch * CHUNK
            pltpu.async_copy(z_hbm.at[ii_v.at[pl.ds(off, CHUNK)]], ri, si)
            pltpu.async_copy(z_hbm.at[jj_v.at[pl.ds(off, CHUNK)]], rj, sj)

        def wait(b):
            ri, rj, si, sj = bufs[b]
            pltpu.make_async_copy(z_hbm.at[ii_v.at[pl.ds(0, CHUNK)]],
                                  ri, si).wait()
            pltpu.make_async_copy(z_hbm.at[jj_v.at[pl.ds(0, CHUNK)]],
                                  rj, sj).wait()

        def compute(ch, b):
            ri, rj, _, _ = bufs[b]
            off = ch * CHUNK

            def group_body(g, _):
                ab_vec = jnp.zeros((L,), jnp.float32)
                aa_vec = jnp.zeros((L,), jnp.float32)
                bb_vec = jnp.zeros((L,), jnp.float32)
                for k in range(L):
                    p = g * L + k
                    ab0 = jnp.zeros((L,), jnp.float32)
                    aa0 = jnp.zeros((L,), jnp.float32)
                    bb0 = jnp.zeros((L,), jnp.float32)
                    ab1 = jnp.zeros((L,), jnp.float32)
                    aa1 = jnp.zeros((L,), jnp.float32)
                    bb1 = jnp.zeros((L,), jnp.float32)
                    for s in range(D // (2 * L)):
                        av0 = ri[p, pl.ds(2 * s * L, L)]
                        bv0 = rj[p, pl.ds(2 * s * L, L)]
                        av1 = ri[p, pl.ds((2 * s + 1) * L, L)]
                        bv1 = rj[p, pl.ds((2 * s + 1) * L, L)]
                        ab0 = ab0 + av0 * bv0
                        aa0 = aa0 + av0 * av0
                        bb0 = bb0 + bv0 * bv0
                        ab1 = ab1 + av1 * bv1
                        aa1 = aa1 + av1 * av1
                        bb1 = bb1 + bv1 * bv1
                    ab_vec = jnp.where(lane == k, jnp.sum(ab0 + ab1), ab_vec)
                    aa_vec = jnp.where(lane == k, jnp.sum(aa0 + aa1), aa_vec)
                    bb_vec = jnp.where(lane == k, jnp.sum(bb0 + bb1), bb_vec)
                na = aa_vec * _fast_rsqrt(aa_vec)
                nb = bb_vec * _fast_rsqrt(bb_vec)
                denom = jnp.maximum(na, EPS) * jnp.maximum(nb, EPS)
                logit = (ab_vec / denom) * TEMP_INV
                accexp[...] += jnp.exp(logit - 10.0)
                gidx = base + off + g * L + lane
                accpos[...] += jnp.where(gidx < n_pos, logit, 0.0)
                return 0

            lax.fori_loop(0, CHUNK // L, group_body, 0)

        issue(0, 0)

        def outer_body(kk, _):
            for b in (0, 1):
                ch = kk * 2 + b

                @pl.when(ch + 1 < nch)
                def _():
                    issue(ch + 1, 1 - b)

                wait(b)
                compute(ch, b)
            return 0

        lax.fori_loop(0, nch // 2, outer_body, 0)
        pltpu.sync_copy(accexp, oexp_hbm.at[wid])
        pltpu.sync_copy(accpos, opos_hbm.at[wid])

    return sc_kernel


def kernel(z_all, pos_pairs, neg_pairs):
    n_pos = pos_pairs.shape[0]
    pairs = jnp.concatenate([pos_pairs, neg_pairs], axis=0)
    ii = pairs[:, 0]
    jj = pairs[:, 1]
    sc = _make_sc_kernel(pairs.shape[0], n_pos)
    part_exp, part_pos = sc(z_all, ii, jj)
    lse = 10.0 + jnp.log(jnp.sum(part_exp))
    return lse - jnp.sum(part_pos) / n_pos
